# baseline (speedup 1.0000x reference)
import functools

import jax
import jax.numpy as jnp
from jax.experimental import pallas as pl
from jax.experimental.pallas import tpu as pltpu

_ROW_TILE = 256
_CHUNK = 2048


def _matmul_kernel(h_ref, w_ref, al_ref, ar_ref, feat_ref, el_ref, er_ref):
    feat = jnp.dot(h_ref[...], w_ref[...], preferred_element_type=jnp.float32)
    feat_ref[...] = feat
    el_ref[...] = jnp.dot(feat, al_ref[...], preferred_element_type=jnp.float32)
    er_ref[...] = jnp.dot(feat, ar_ref[...], preferred_element_type=jnp.float32)


def _edge_kernel(src_ref, dst_ref, feat_ref, el_ref, er_ref, bias_ref,
                 out_ref, s_ref, *, n_chunks, chunk, heads, dim):
    phase = pl.program_id(0)
    j = pl.program_id(1)
    hd = heads * dim

    @pl.when(jnp.logical_and(phase == 0, j == 0))
    def _init():
        s_ref[...] = jnp.zeros_like(s_ref)
        out_ref[...] = jnp.zeros_like(out_ref)

    @pl.when(phase == 0)
    def _pass1():
        def body(i, carry):
            sidx = src_ref[i]
            didx = dst_ref[i]
            ev = el_ref[pl.ds(sidx, 1), :] + er_ref[pl.ds(didx, 1), :]
            ev = jnp.where(ev >= 0.0, ev, 0.2 * ev)
            s_ref[pl.ds(didx, 1), :] += jnp.exp(ev)
            return carry

        jax.lax.fori_loop(0, chunk, body, 0)

    @pl.when(phase == 1)
    def _pass2():
        lane = jax.lax.broadcasted_iota(jnp.int32, (heads, hd), 1)
        head = jax.lax.broadcasted_iota(jnp.int32, (heads, hd), 0)
        expand = (lane // dim == head).astype(jnp.float32)

        def body(i, carry):
            sidx = src_ref[i]
            didx = dst_ref[i]
            ev = el_ref[pl.ds(sidx, 1), :] + er_ref[pl.ds(didx, 1), :]
            ev = jnp.where(ev >= 0.0, ev, 0.2 * ev)
            alpha = jnp.exp(ev) / (s_ref[pl.ds(didx, 1), :] + 1e-9)
            a512 = jnp.dot(alpha, expand, preferred_element_type=jnp.float32)
            out_ref[pl.ds(didx, 1), :] += a512 * feat_ref[pl.ds(sidx, 1), :]
            return carry

        jax.lax.fori_loop(0, chunk, body, 0)

    @pl.when(jnp.logical_and(phase == 1, j == n_chunks - 1))
    def _finish():
        x = out_ref[...] + bias_ref[...]
        out_ref[...] = jnp.where(x > 0.0, x, jnp.exp(jnp.minimum(x, 0.0)) - 1.0)


def kernel(h, edge_index, W, attn_l, attn_r, bias, W1, b1, W2):
    n, in_size = h.shape
    heads, dim = attn_l.shape
    hd = heads * dim
    e = edge_index.shape[1]

    eye = jnp.eye(heads, dtype=jnp.float32)
    al = (attn_l[:, :, None] * eye[:, None, :]).reshape(hd, heads)
    ar = (attn_r[:, :, None] * eye[:, None, :]).reshape(hd, heads)

    n_pad = ((n + _ROW_TILE - 1) // _ROW_TILE) * _ROW_TILE
    h_p = jnp.pad(h, ((0, n_pad - n), (0, 0)))

    feat_p, el_p, er_p = pl.pallas_call(
        _matmul_kernel,
        grid=(n_pad // _ROW_TILE,),
        in_specs=[
            pl.BlockSpec((_ROW_TILE, in_size), lambda i: (i, 0)),
            pl.BlockSpec((in_size, hd), lambda i: (0, 0)),
            pl.BlockSpec((hd, heads), lambda i: (0, 0)),
            pl.BlockSpec((hd, heads), lambda i: (0, 0)),
        ],
        out_specs=[
            pl.BlockSpec((_ROW_TILE, hd), lambda i: (i, 0)),
            pl.BlockSpec((_ROW_TILE, heads), lambda i: (i, 0)),
            pl.BlockSpec((_ROW_TILE, heads), lambda i: (i, 0)),
        ],
        out_shape=[
            jax.ShapeDtypeStruct((n_pad, hd), jnp.float32),
            jax.ShapeDtypeStruct((n_pad, heads), jnp.float32),
            jax.ShapeDtypeStruct((n_pad, heads), jnp.float32),
        ],
    )(h_p, W, al, ar)

    feat = feat_p[:n]
    el = el_p[:n]
    er = er_p[:n]

    n_chunks = (e + _CHUNK - 1) // _CHUNK
    e_pad = n_chunks * _CHUNK
    src = jnp.pad(edge_index[0], (0, e_pad - e))
    dst = jnp.pad(edge_index[1], (0, e_pad - e), constant_values=n)
    acc_rows = n + 8

    out = pl.pallas_call(
        functools.partial(_edge_kernel, n_chunks=n_chunks, chunk=_CHUNK,
                          heads=heads, dim=dim),
        grid=(2, n_chunks),
        in_specs=[
            pl.BlockSpec((_CHUNK,), lambda p, j: (j,),
                         memory_space=pltpu.SMEM),
            pl.BlockSpec((_CHUNK,), lambda p, j: (j,),
                         memory_space=pltpu.SMEM),
            pl.BlockSpec((n, hd), lambda p, j: (0, 0)),
            pl.BlockSpec((n, heads), lambda p, j: (0, 0)),
            pl.BlockSpec((n, heads), lambda p, j: (0, 0)),
            pl.BlockSpec((1, hd), lambda p, j: (0, 0)),
        ],
        out_specs=pl.BlockSpec((acc_rows, hd), lambda p, j: (0, 0)),
        out_shape=jax.ShapeDtypeStruct((acc_rows, hd), jnp.float32),
        scratch_shapes=[
            pltpu.VMEM((acc_rows, heads), jnp.float32),
        ],
    )(src, dst, feat, el, er, bias.reshape(1, hd))

    emb = out[:n]
    return emb
